# baseline (speedup 1.0000x reference)
import functools

import jax
import jax.numpy as jnp
from jax import lax
from jax.experimental import pallas as pl
from jax.experimental.pallas import tpu as pltpu
from jax.experimental.pallas import tpu_sc as plsc

XS, YS = 432, 496
CELLS = XS * YS
NP, NF = 25000, 64
NPAD = 25088
PT = NPAD // 16
NV = PT // 16
CH = 26784
NCH = 8
CPAD = NCH * CH
OUT_PT = CH // 16
ZPT = (CH + 16) // 16
G = 64
LCAP = PT + G + 16
DCH = CELLS + 16
DPT = DCH // 16
SENT = 2147483647


def _dyn_tc_body(p_ref, w_ref, b_ref, o_ref):
    x = p_ref[...]
    logits = lax.dot_general(x, w_ref[...], (((1,), (1,)), ((), ())),
                             preferred_element_type=jnp.float32)
    logits = logits + b_ref[...]
    m = jnp.max(logits, axis=1, keepdims=True)
    e = jnp.exp(logits - m)
    sm = e / jnp.sum(e, axis=1, keepdims=True)
    o_ref[...] = jnp.pad(sm, ((0, 0), (0, 5)))


def _dyn_tc(pil2, W, b):
    return pl.pallas_call(
        _dyn_tc_body,
        grid=(10,),
        in_specs=[
            pl.BlockSpec((5000, NF), lambda gi: (gi, 0)),
            pl.BlockSpec((3, NF), lambda gi: (0, 0)),
            pl.BlockSpec((1, 3), lambda gi: (0, 0)),
        ],
        out_specs=pl.BlockSpec((5000, 8), lambda gi: (gi, 0)),
        out_shape=jax.ShapeDtypeStruct((2 * NP, 8), jnp.float32),
    )(pil2, W, b.reshape(1, 3))


RC = 3456


def _xpose_body(p_ref, o_ref):
    o_ref[0] = p_ref[0].T


def _xpose_tc(outp):
    x = outp.reshape(2, CELLS, NF)
    return pl.pallas_call(
        _xpose_body,
        grid=(2, CELLS // RC),
        in_specs=[pl.BlockSpec((1, RC, NF), lambda bi, gi: (bi, gi, 0))],
        out_specs=pl.BlockSpec((1, NF, RC), lambda bi, gi: (bi, 0, gi)),
        out_shape=jax.ShapeDtypeStruct((2, NF, CELLS), jnp.float32),
    )(x)


def _load_indices(yh, xh, ch, yv, xv, cv, idxt, base, sentinel):
    pltpu.sync_copy(yh.at[pl.ds(base, PT)], yv)
    pltpu.sync_copy(xh.at[pl.ds(base, PT)], xv)
    pltpu.sync_copy(ch.at[pl.ds(base, PT)], cv)

    def cidx(j, _):
        iv = yv[pl.ds(16 * j, 16)] * XS + xv[pl.ds(16 * j, 16)]
        valid = cv[pl.ds(16 * j, 16)] == 1
        idxt[pl.ds(16 * j, 16)] = jnp.where(valid, iv, sentinel)
        return 0

    lax.fori_loop(0, NV, cidx, 0)


def _compact(idxt, srcl, locl, lanes, lo, hi, srcbase, padsrc, padloc):
    def scan(j, cnt):
        v = idxt[pl.ds(16 * j, 16)]
        m = (v >= lo) & (v < hi)
        src = srcbase + 16 * j + lanes
        mi = m.astype(jnp.int32)
        pos = cnt + plsc.cumsum(mi) - 1
        plsc.store_scatter(srcl, [pos], src, mask=m)
        plsc.store_scatter(locl, [pos], v - lo, mask=m)
        return cnt + jnp.sum(mi)

    cnt = lax.fori_loop(0, NV, scan, jnp.int32(0))
    psrc = jnp.zeros((16,), jnp.int32) + padsrc
    ploc = jnp.zeros((16,), jnp.int32) + padloc
    for k in range(G // 16):
        srcl[pl.ds(cnt + 16 * k, 16)] = psrc
        locl[pl.ds(cnt + 16 * k, 16)] = ploc
    return (cnt + G - 1) // G


def _stage_idx(stage, lst, g):
    for k in range(G // 16):
        stage[0, pl.ds(16 * k, 16)] = lst[pl.ds(g * G + 16 * k, 16)]


def _zero_stag(stag):
    def zr(i, _):
        for k in range(NF // 16):
            stag[i, pl.ds(16 * k, 16)] = jnp.zeros((16,), jnp.float32)
        return 0

    lax.fori_loop(0, G, zr, 0)


def _pseudo_body(pil2, yh, xh, ch, out,
                 acc, yv, xv, cv, idxt, srcl, locl, sstage, lstage, stag, stag2):
    c = lax.axis_index("c")
    s = lax.axis_index("s")
    lanes = lax.iota(jnp.int32, 16)
    _load_indices(yh, xh, ch, yv, xv, cv, idxt, c * NPAD + s * PT,
                  jnp.int32(SENT))
    _zero_stag(stag)
    for k in range(ZPT // G):
        pltpu.sync_copy(stag, acc.at[pl.ds(s * ZPT + k * G, G)])
    pltpu.sync_copy(stag.at[pl.ds(0, ZPT % G)],
                    acc.at[pl.ds(s * ZPT + (ZPT // G) * G, ZPT % G)])
    plsc.subcore_barrier()

    def chunk(i, _):
        lo = i * CH
        ng = _compact(idxt, srcl, locl, lanes, lo, lo + CH,
                      c * NP + s * PT, c * NP + s, CH + s)

        def grp(g, _):
            _stage_idx(sstage, srcl, g)
            _stage_idx(lstage, locl, g)
            pltpu.sync_copy(pil2.at[sstage.at[0]], stag2)
            pltpu.sync_copy(stag2, acc.at[lstage.at[0]], add=True)
            return 0

        lax.fori_loop(0, ng, grp, 0)
        plsc.subcore_barrier()
        pltpu.sync_copy(acc.at[pl.ds(s * OUT_PT, OUT_PT)],
                        out.at[pl.ds(c * CPAD + lo + s * OUT_PT, OUT_PT)])
        plsc.subcore_barrier()
        _zero_stag(stag)

        def rz(g, _):
            _stage_idx(lstage, locl, g)
            pltpu.sync_copy(stag, acc.at[lstage.at[0]])
            return 0

        lax.fori_loop(0, ng, rz, 0)
        plsc.subcore_barrier()
        return 0

    lax.fori_loop(0, NCH, chunk, 0)


def _dyn_sc_body(dyn, yh, xh, ch, zh, out,
                 acc, yv, xv, cv, idxt, srcl, locl, sstage, lstage, stag, zstag):
    c = lax.axis_index("c")
    s = lax.axis_index("s")
    lanes = lax.iota(jnp.int32, 16)
    _load_indices(yh, xh, ch, yv, xv, cv, idxt, c * NPAD + s * PT,
                  jnp.int32(SENT))
    pltpu.sync_copy(zh, zstag)
    for k in range(26):
        pltpu.sync_copy(zstag, acc.at[pl.ds(s * DPT + k * 512, 512)])
    pltpu.sync_copy(zstag.at[pl.ds(0, DPT - 26 * 512)],
                    acc.at[pl.ds(s * DPT + 26 * 512, DPT - 26 * 512)])
    plsc.subcore_barrier()

    ng = _compact(idxt, srcl, locl, lanes, jnp.int32(0), jnp.int32(CELLS),
                  c * NP + s * PT, c * NP + s, CELLS + s)

    def grp(g, _):
        _stage_idx(sstage, srcl, g)
        _stage_idx(lstage, locl, g)
        pltpu.sync_copy(dyn.at[sstage.at[0]], stag)
        pltpu.sync_copy(stag, acc.at[lstage.at[0]], add=True)
        return 0

    lax.fori_loop(0, ng, grp, 0)
    plsc.subcore_barrier()
    pltpu.sync_copy(acc.at[pl.ds(s * DPT, DPT)],
                    out.at[pl.ds(c * DCH + s * DPT, DPT)])


def _sc_mesh():
    return plsc.VectorSubcoreMesh(core_axis_name="c", subcore_axis_name="s",
                                  num_cores=2, num_subcores=16)


def _idx_scratches():
    return [
        pltpu.VMEM((PT,), jnp.int32),
        pltpu.VMEM((PT,), jnp.int32),
        pltpu.VMEM((PT,), jnp.int32),
        pltpu.VMEM((PT,), jnp.int32),
        pltpu.VMEM((LCAP,), jnp.int32),
        pltpu.VMEM((LCAP,), jnp.int32),
        pltpu.VMEM((1, G), jnp.int32),
        pltpu.VMEM((1, G), jnp.int32),
    ]


def kernel(pillars, coord, contains_pillars, W, b):
    y = jnp.pad(coord[:, :, 1].astype(jnp.int32), ((0, 0), (0, NPAD - NP)))
    x = jnp.pad(coord[:, :, 2].astype(jnp.int32), ((0, 0), (0, NPAD - NP)))
    cont = jnp.pad(contains_pillars.astype(jnp.int32), ((0, 0), (0, NPAD - NP)))
    yf, xf, cf = y.reshape(-1), x.reshape(-1), cont.reshape(-1)
    pil2 = pillars.reshape(2 * NP, NF)

    dyn4 = _dyn_tc(pil2, W, b)

    outp = pl.kernel(
        _pseudo_body,
        out_type=jax.ShapeDtypeStruct((2 * CPAD, NF), jnp.float32),
        mesh=_sc_mesh(),
        compiler_params=pltpu.CompilerParams(needs_layout_passes=False, use_tc_tiling_on_sc=False),
        scratch_types=[pltpu.VMEM_SHARED((CH + 16, NF), jnp.float32)]
        + _idx_scratches()
        + [pltpu.VMEM((G, NF), jnp.float32),
           pltpu.VMEM((G, NF), jnp.float32)],
    )(pil2, yf, xf, cf)

    zeros_d = jnp.zeros((512, 8), jnp.float32)
    outd = pl.kernel(
        _dyn_sc_body,
        out_type=jax.ShapeDtypeStruct((2 * DCH, 8), jnp.float32),
        mesh=_sc_mesh(),
        compiler_params=pltpu.CompilerParams(needs_layout_passes=False, use_tc_tiling_on_sc=False),
        scratch_types=[pltpu.VMEM_SHARED((DCH, 8), jnp.float32)]
        + _idx_scratches()
        + [pltpu.VMEM((G, 8), jnp.float32),
           pltpu.VMEM((512, 8), jnp.float32)],
    )(dyn4, yf, xf, cf, zeros_d)

    pseudo = (outp.reshape(2, CPAD, NF)[:, :CELLS]
              .reshape(2, XS, YS, NF).transpose(0, 3, 1, 2))
    dynamic = (outd.reshape(2, DCH, 8)[:, :CELLS, :3]
               .reshape(2, XS, YS, 3).transpose(0, 3, 1, 2))
    return pseudo, dynamic


# speedup vs baseline: 1.0631x; 1.0631x over previous
import jax
import jax.numpy as jnp
from jax import lax
from jax.experimental import pallas as pl
from jax.experimental.pallas import tpu as pltpu
from jax.experimental.pallas import tpu_sc as plsc

XS, YS = 432, 496
CELLS = XS * YS
NP, NF = 25000, 64
NPAD = 25088
PT = NPAD // 16
NV = PT // 16
CH = 26784
NCH = 8
CPAD = NCH * CH
OUT_PT = CH // 16
ZPT = (CH + 16) // 16
G = 64
LCAP = PT + G + 16
DCH = CELLS + 16
DPT = DCH // 16
SENT = 2147483647


def _dyn_tc_body(p_ref, w_ref, b_ref, o_ref):
    x = p_ref[...]
    logits = lax.dot_general(x, w_ref[...], (((1,), (1,)), ((), ())),
                             preferred_element_type=jnp.float32)
    logits = logits + b_ref[...]
    m = jnp.max(logits, axis=1, keepdims=True)
    e = jnp.exp(logits - m)
    sm = e / jnp.sum(e, axis=1, keepdims=True)
    o_ref[...] = jnp.pad(sm, ((0, 0), (0, 5)))


def _dyn_tc(pil2, W, b):
    return pl.pallas_call(
        _dyn_tc_body,
        grid=(10,),
        in_specs=[
            pl.BlockSpec((5000, NF), lambda gi: (gi, 0)),
            pl.BlockSpec((3, NF), lambda gi: (0, 0)),
            pl.BlockSpec((1, 3), lambda gi: (0, 0)),
        ],
        out_specs=pl.BlockSpec((5000, 8), lambda gi: (gi, 0)),
        out_shape=jax.ShapeDtypeStruct((2 * NP, 8), jnp.float32),
    )(pil2, W, b.reshape(1, 3))


def _load_indices(yh, xh, ch, yv, xv, cv, idxt, base, sentinel):
    pltpu.sync_copy(yh.at[pl.ds(base, PT)], yv)
    pltpu.sync_copy(xh.at[pl.ds(base, PT)], xv)
    pltpu.sync_copy(ch.at[pl.ds(base, PT)], cv)

    def cidx(j, _):
        iv = yv[pl.ds(16 * j, 16)] * XS + xv[pl.ds(16 * j, 16)]
        valid = cv[pl.ds(16 * j, 16)] == 1
        idxt[pl.ds(16 * j, 16)] = jnp.where(valid, iv, sentinel)
        return 0

    lax.fori_loop(0, NV, cidx, 0)


def _compact(idxt, srcl, locl, lanes, lo, hi, srcbase, padsrc, padloc):
    def scan(j, cnt):
        v = idxt[pl.ds(16 * j, 16)]
        m = (v >= lo) & (v < hi)
        src = srcbase + 16 * j + lanes
        mi = m.astype(jnp.int32)
        pos = cnt + plsc.cumsum(mi) - 1
        plsc.store_scatter(srcl, [pos], src, mask=m)
        plsc.store_scatter(locl, [pos], v - lo, mask=m)
        return cnt + jnp.sum(mi)

    cnt = lax.fori_loop(0, NV, scan, jnp.int32(0))
    psrc = jnp.zeros((16,), jnp.int32) + padsrc
    ploc = jnp.zeros((16,), jnp.int32) + padloc
    for k in range(G // 16):
        srcl[pl.ds(cnt + 16 * k, 16)] = psrc
        locl[pl.ds(cnt + 16 * k, 16)] = ploc
    return (cnt + G - 1) // G


def _stage_idx(stage, lst, g):
    for k in range(G // 16):
        stage[0, pl.ds(16 * k, 16)] = lst[pl.ds(g * G + 16 * k, 16)]


def _zero_stag(stag):
    def zr(i, _):
        for k in range(NF // 16):
            stag[i, pl.ds(16 * k, 16)] = jnp.zeros((16,), jnp.float32)
        return 0

    lax.fori_loop(0, G, zr, 0)


def _pseudo_body(pil2, yh, xh, ch, out,
                 acc, yv, xv, cv, idxt, srcl, locl, sstage, lstage, stag, stag2):
    c = lax.axis_index("c")
    s = lax.axis_index("s")
    lanes = lax.iota(jnp.int32, 16)
    _load_indices(yh, xh, ch, yv, xv, cv, idxt, c * NPAD + s * PT,
                  jnp.int32(SENT))
    _zero_stag(stag)
    for k in range(ZPT // G):
        pltpu.sync_copy(stag, acc.at[pl.ds(s * ZPT + k * G, G)])
    pltpu.sync_copy(stag.at[pl.ds(0, ZPT % G)],
                    acc.at[pl.ds(s * ZPT + (ZPT // G) * G, ZPT % G)])
    plsc.subcore_barrier()

    def chunk(i, _):
        lo = i * CH
        ng = _compact(idxt, srcl, locl, lanes, lo, lo + CH,
                      c * NP + s * PT, c * NP + s, CH + s)

        def grp(g, _):
            _stage_idx(sstage, srcl, g)
            _stage_idx(lstage, locl, g)
            pltpu.sync_copy(pil2.at[sstage.at[0]], stag2)
            pltpu.sync_copy(stag2, acc.at[lstage.at[0]], add=True)
            return 0

        lax.fori_loop(0, ng, grp, 0)
        plsc.subcore_barrier()
        pltpu.sync_copy(acc.at[pl.ds(s * OUT_PT, OUT_PT)],
                        out.at[pl.ds(c * CPAD + lo + s * OUT_PT, OUT_PT)])
        plsc.subcore_barrier()
        _zero_stag(stag)

        def rz(g, _):
            _stage_idx(lstage, locl, g)
            pltpu.sync_copy(stag, acc.at[lstage.at[0]])
            return 0

        lax.fori_loop(0, ng, rz, 0)
        plsc.subcore_barrier()
        return 0

    lax.fori_loop(0, NCH, chunk, 0)


def _dyn_sc_body(dyn, yh, xh, ch, zh, out,
                 acc, yv, xv, cv, idxt, srcl, locl, sstage, lstage, stag, zstag):
    c = lax.axis_index("c")
    s = lax.axis_index("s")
    lanes = lax.iota(jnp.int32, 16)
    _load_indices(yh, xh, ch, yv, xv, cv, idxt, c * NPAD + s * PT,
                  jnp.int32(SENT))
    pltpu.sync_copy(zh, zstag)
    for k in range(26):
        pltpu.sync_copy(zstag, acc.at[pl.ds(s * DPT + k * 512, 512)])
    pltpu.sync_copy(zstag.at[pl.ds(0, DPT - 26 * 512)],
                    acc.at[pl.ds(s * DPT + 26 * 512, DPT - 26 * 512)])
    plsc.subcore_barrier()

    ng = _compact(idxt, srcl, locl, lanes, jnp.int32(0), jnp.int32(CELLS),
                  c * NP + s * PT, c * NP + s, CELLS + s)

    def grp(g, _):
        _stage_idx(sstage, srcl, g)
        _stage_idx(lstage, locl, g)
        pltpu.sync_copy(dyn.at[sstage.at[0]], stag)
        pltpu.sync_copy(stag, acc.at[lstage.at[0]], add=True)
        return 0

    lax.fori_loop(0, ng, grp, 0)
    plsc.subcore_barrier()
    pltpu.sync_copy(acc.at[pl.ds(s * DPT, DPT)],
                    out.at[pl.ds(c * DCH + s * DPT, DPT)])


def _sc_mesh():
    return plsc.VectorSubcoreMesh(core_axis_name="c", subcore_axis_name="s",
                                  num_cores=2, num_subcores=16)


def _idx_scratches():
    return [
        pltpu.VMEM((PT,), jnp.int32),
        pltpu.VMEM((PT,), jnp.int32),
        pltpu.VMEM((PT,), jnp.int32),
        pltpu.VMEM((PT,), jnp.int32),
        pltpu.VMEM((LCAP,), jnp.int32),
        pltpu.VMEM((LCAP,), jnp.int32),
        pltpu.VMEM((1, G), jnp.int32),
        pltpu.VMEM((1, G), jnp.int32),
    ]


def kernel(pillars, coord, contains_pillars, W, b):
    y = jnp.pad(coord[:, :, 1].astype(jnp.int32), ((0, 0), (0, NPAD - NP)))
    x = jnp.pad(coord[:, :, 2].astype(jnp.int32), ((0, 0), (0, NPAD - NP)))
    cont = jnp.pad(contains_pillars.astype(jnp.int32), ((0, 0), (0, NPAD - NP)))
    yf, xf, cf = y.reshape(-1), x.reshape(-1), cont.reshape(-1)
    pil2 = pillars.reshape(2 * NP, NF)

    dyn4 = _dyn_tc(pil2, W, b)

    outp = pl.kernel(
        _pseudo_body,
        out_type=jax.ShapeDtypeStruct((2 * CPAD, NF), jnp.float32),
        mesh=_sc_mesh(),
        compiler_params=pltpu.CompilerParams(needs_layout_passes=False, use_tc_tiling_on_sc=False),
        scratch_types=[pltpu.VMEM_SHARED((CH + 16, NF), jnp.float32)]
        + _idx_scratches()
        + [pltpu.VMEM((G, NF), jnp.float32),
           pltpu.VMEM((G, NF), jnp.float32)],
    )(pil2, yf, xf, cf)

    zeros_d = jnp.zeros((512, 8), jnp.float32)
    outd = pl.kernel(
        _dyn_sc_body,
        out_type=jax.ShapeDtypeStruct((2 * DCH, 8), jnp.float32),
        mesh=_sc_mesh(),
        compiler_params=pltpu.CompilerParams(needs_layout_passes=False, use_tc_tiling_on_sc=False),
        scratch_types=[pltpu.VMEM_SHARED((DCH, 8), jnp.float32)]
        + _idx_scratches()
        + [pltpu.VMEM((G, 8), jnp.float32),
           pltpu.VMEM((512, 8), jnp.float32)],
    )(dyn4, yf, xf, cf, zeros_d)

    pseudo = (outp.reshape(2, CPAD, NF)[:, :CELLS]
              .reshape(2, XS, YS, NF).transpose(0, 3, 1, 2))
    dynamic = (outd.reshape(2, DCH, 8)[:, :CELLS, :3]
               .reshape(2, XS, YS, 3).transpose(0, 3, 1, 2))
    return pseudo, dynamic
